# baseline (speedup 1.0000x reference)
import jax
import jax.numpy as jnp
from jax.experimental import pallas as pl
from jax.experimental.pallas import tpu as pltpu

N, I, O, B, T = 2048, 1024, 1024, 8, 16
OC = 256
NSTEP = O // OC


def _fused_kernel(emb_ref, g_ref, x_ref, w_ref, bb_ref,
                  out_ref, logp_ref, probs_ref, h_ref, x16_ref):
    step = pl.program_id(0)

    @pl.when(step == 0)
    def _init():
        emb = emb_ref[...]
        iota_t = jax.lax.broadcasted_iota(jnp.int32, (N, T), 1)
        onehot = (emb == iota_t).astype(jnp.float32)
        g_sel = jnp.dot(onehot, g_ref[...],
                        preferred_element_type=jnp.float32)
        g_max = jnp.max(g_sel, axis=-1, keepdims=True)
        e = jnp.exp(g_sel - g_max)
        probs = e / jnp.sum(e, axis=-1, keepdims=True)
        probs_ref[...] = probs
        logp_ref[...] = jnp.log(jnp.max(probs, axis=-1, keepdims=True) + 1e-9)
        x16_ref[...] = x_ref[...].astype(jnp.bfloat16)

    x = x16_ref[...]
    w = w_ref[...].reshape(B * OC, I).astype(jnp.bfloat16)
    h_ref[...] = jax.lax.dot_general(
        x, w, (((1,), (1,)), ((), ())),
        preferred_element_type=jnp.float32).astype(jnp.bfloat16)

    probs = probs_ref[...]
    acc = jnp.zeros((N, OC), jnp.float32)
    for b in range(B):
        pb = probs[:, b:b + 1]
        hb = (h_ref[:, b * OC:(b + 1) * OC].astype(jnp.float32)
              + bb_ref[b:b + 1, :])
        acc = acc + pb * hb
    out_ref[...] = acc


def kernel(iput, emb, weights, g_logits, W_blocks, b_blocks):
    emb = emb.astype(jnp.int32)

    out, logp = pl.pallas_call(
        _fused_kernel,
        grid=(NSTEP,),
        in_specs=[
            pl.BlockSpec((N, 1), lambda s: (0, 0)),
            pl.BlockSpec((T, B), lambda s: (0, 0)),
            pl.BlockSpec((N, I), lambda s: (0, 0)),
            pl.BlockSpec((B, OC, I), lambda s: (0, s, 0)),
            pl.BlockSpec((B, OC), lambda s: (0, s)),
        ],
        out_specs=[
            pl.BlockSpec((N, OC), lambda s: (0, s)),
            pl.BlockSpec((N, 1), lambda s: (0, 0)),
        ],
        out_shape=[
            jax.ShapeDtypeStruct((N, O), jnp.float32),
            jax.ShapeDtypeStruct((N, 1), jnp.float32),
        ],
        scratch_shapes=[
            pltpu.VMEM((N, B), jnp.float32),
            pltpu.VMEM((N, B * OC), jnp.bfloat16),
            pltpu.VMEM((N, I), jnp.bfloat16),
        ],
    )(emb, g_logits, iput, W_blocks, b_blocks)

    return out, logp.reshape(N), jnp.float32(0.0)


# speedup vs baseline: 1.0593x; 1.0593x over previous
import jax
import jax.numpy as jnp
from jax.experimental import pallas as pl
from jax.experimental.pallas import tpu as pltpu

N, I, O, B, T = 2048, 1024, 1024, 8, 16
OC = 128
NSTEP = O // OC


def _fused_kernel(emb_ref, g_ref, x_ref, w_ref, bb_ref,
                  out_ref, logp_ref, probs_ref, h_ref, x16_ref):
    step = pl.program_id(0)

    @pl.when(step == 0)
    def _init():
        emb = emb_ref[...]
        iota_t = jax.lax.broadcasted_iota(jnp.int32, (N, T), 1)
        onehot = (emb == iota_t).astype(jnp.float32)
        g_sel = jnp.dot(onehot, g_ref[...],
                        preferred_element_type=jnp.float32)
        g_max = jnp.max(g_sel, axis=-1, keepdims=True)
        e = jnp.exp(g_sel - g_max)
        probs = e / jnp.sum(e, axis=-1, keepdims=True)
        probs_ref[...] = probs
        logp_ref[...] = jnp.log(jnp.max(probs, axis=-1, keepdims=True) + 1e-9)
        x16_ref[...] = x_ref[...].astype(jnp.bfloat16)

    x = x16_ref[...]
    w = w_ref[...].reshape(B * OC, I).astype(jnp.bfloat16)
    h_ref[...] = jax.lax.dot_general(
        x, w, (((1,), (1,)), ((), ())),
        preferred_element_type=jnp.float32)

    probs = probs_ref[...]
    acc = jnp.zeros((N, OC), jnp.float32)
    for b in range(B):
        pb = probs[:, b:b + 1]
        hb = h_ref[:, b * OC:(b + 1) * OC] + bb_ref[b:b + 1, :]
        acc = acc + pb * hb
    out_ref[...] = acc


def kernel(iput, emb, weights, g_logits, W_blocks, b_blocks):
    emb = emb.astype(jnp.int32)

    out, logp = pl.pallas_call(
        _fused_kernel,
        grid=(NSTEP,),
        in_specs=[
            pl.BlockSpec((N, 1), lambda s: (0, 0)),
            pl.BlockSpec((T, B), lambda s: (0, 0)),
            pl.BlockSpec((N, I), lambda s: (0, 0)),
            pl.BlockSpec((B, OC, I), lambda s: (0, s, 0)),
            pl.BlockSpec((B, OC), lambda s: (0, s)),
        ],
        out_specs=[
            pl.BlockSpec((N, OC), lambda s: (0, s)),
            pl.BlockSpec((N, 1), lambda s: (0, 0)),
        ],
        out_shape=[
            jax.ShapeDtypeStruct((N, O), jnp.float32),
            jax.ShapeDtypeStruct((N, 1), jnp.float32),
        ],
        scratch_shapes=[
            pltpu.VMEM((N, B), jnp.float32),
            pltpu.VMEM((N, B * OC), jnp.float32),
            pltpu.VMEM((N, I), jnp.bfloat16),
        ],
    )(emb, g_logits, iput, W_blocks, b_blocks)

    return out, logp.reshape(N), jnp.float32(0.0)
